# Initial kernel scaffold; baseline (speedup 1.0000x reference)
import jax
import jax.numpy as jnp
from jax.experimental import pallas as pl


def kernel(x, pos_table):
    raise NotImplementedError("write your pallas kernel here")


# speedup vs baseline: 2.3656x; 2.3656x over previous
import jax
import jax.numpy as jnp
from jax.experimental import pallas as pl


def _add_block(x_ref, t_ref, o_ref):
    o_ref[...] = x_ref[...] + t_ref[...]


def kernel(x, pos_table):
    n, d = x.shape
    table = pos_table[:n]
    block_rows = 512
    grid = (n // block_rows,)
    spec = pl.BlockSpec((block_rows, d), lambda i: (i, 0))
    return pl.pallas_call(
        _add_block,
        grid=grid,
        in_specs=[spec, spec],
        out_specs=spec,
        out_shape=jax.ShapeDtypeStruct((n, d), x.dtype),
    )(x, table)
